# baseline (speedup 1.0000x reference)
import functools

import jax
import jax.numpy as jnp
from jax import lax
from jax.experimental import pallas as pl
from jax.experimental.pallas import tpu as pltpu
from jax.experimental.pallas import tpu_sc as plsc


def _triplet_indices(target):
    B = target.shape[0]
    idx = jnp.arange(B, dtype=jnp.int32)
    t = target.astype(jnp.int32)
    skey = jnp.sort(t * B + idx)
    order = skey % B
    sorted_lbl = skey // B
    new_group = jnp.concatenate(
        [jnp.array([True]), sorted_lbl[1:] != sorted_lbl[:-1]])
    starts_per_pos = jax.lax.cummax(jnp.where(new_group, idx, 0))
    flagged = jnp.where(new_group, idx, B)
    rev_min = jax.lax.cummin(flagged, reverse=True)
    next_start = jnp.concatenate([rev_min[1:], jnp.array([B], rev_min.dtype)])
    counts = next_start - starts_per_pos
    pos_within = idx - starts_per_pos
    partner_sorted = starts_per_pos + (pos_within + 1) % counts
    partner = order[partner_sorted]
    positive = jnp.zeros(B, jnp.int32).at[order].set(partner)
    inv = jnp.zeros(B, jnp.int32).at[partner].set(order)
    l0 = t[0]
    diff = t != l0
    j1 = jnp.where(jnp.any(diff), jnp.argmax(diff).astype(jnp.int32),
                   jnp.int32(-1))
    valid = (positive != idx) & (diff | (j1 >= 0))
    return positive, inv, valid, j1, l0


def _make_sc_scatter(B, C, n_workers, bands_per_w):
    n_full = C // 128
    mesh = plsc.VectorSubcoreMesh(core_axis_name="core",
                                  subcore_axis_name="subcore")

    @functools.partial(
        pl.kernel,
        out_type=(jax.ShapeDtypeStruct((B * 8, 128), jnp.float32),
                  jax.ShapeDtypeStruct((B * 8, 128), jnp.float32)),
        mesh=mesh,
        scratch_types=[
            pltpu.VMEM((bands_per_w, 64), jnp.int32),
            pltpu.VMEM((bands_per_w, 64), jnp.int32),
            pltpu.VMEM((2, 64, 128), jnp.float32),
            pltpu.SemaphoreType.DMA,
        ],
    )
    def sc_kernel(x_hbm, xs_hbm, ia_hbm, iy_hbm, oa_hbm, oy_hbm,
                  idxa_v, idxy_v, buf, gsem):
        wid = lax.axis_index("subcore") * 2 + lax.axis_index("core")
        pltpu.sync_copy(ia_hbm.at[wid], idxa_v)
        pltpu.sync_copy(iy_hbm.at[wid], idxy_v)

        def start_band(bl, slot):
            b = wid * bands_per_w + bl
            for t in range(n_full):
                pltpu.make_async_copy(
                    x_hbm.at[pl.ds(b * 8, 8), pl.ds(t * 128, 128)],
                    buf.at[slot, pl.ds(t * 8, 8), :], gsem).start()
            pltpu.make_async_copy(
                xs_hbm.at[pl.ds(b * 8, 8), :],
                buf.at[slot, pl.ds(n_full * 8, 8), :], gsem).start()

        def wait_band(slot):
            for t in range(n_full + 1):
                pltpu.make_async_copy(
                    x_hbm.at[pl.ds(0, 8), pl.ds(0, 128)],
                    buf.at[slot, pl.ds(t * 8, 8), :], gsem).wait()

        start_band(0, 0)

        @pl.loop(0, bands_per_w, step=2)
        def _(c):
            for par in range(2):
                slot = par
                other = 1 - par
                bl = c + par

                @pl.when(bl + 1 < bands_per_w)
                def _():
                    start_band(bl + 1, other)

                wait_band(slot)
                pltpu.sync_copy(buf.at[slot], oa_hbm.at[idxa_v.at[bl]])
                pltpu.sync_copy(buf.at[slot], oy_hbm.at[idxy_v.at[bl]])

    return sc_kernel


def _loss2d_body(meta_ref, pos_ref, x_any, x_blk, tgt_ref, valid_ref, out_ref,
                 gbuf, negrows, acc, gsem, nsem, *, rows, cols):
    i = pl.program_id(0)
    nsteps = pl.num_programs(0)
    slot = jax.lax.rem(i, 2)
    nxt = 1 - slot

    @pl.when(i == 0)
    def _init():
        acc[0] = 0.0
        acc[1] = 0.0
        pltpu.make_async_copy(x_any.at[pl.ds(0, 1), :],
                              negrows.at[pl.ds(0, 1), :], nsem).start()
        pltpu.make_async_copy(x_any.at[pl.ds(meta_ref[0], 1), :],
                              negrows.at[pl.ds(1, 1), :], nsem).start()
        for r in range(rows):
            pltpu.make_async_copy(
                x_any.at[pl.ds(pos_ref[r], 1), :],
                gbuf.at[slot, pl.ds(r, 1), :], gsem).start()
        pltpu.make_async_copy(x_any.at[pl.ds(0, 1), :],
                              negrows.at[pl.ds(0, 1), :], nsem).wait()
        pltpu.make_async_copy(x_any.at[pl.ds(0, 1), :],
                              negrows.at[pl.ds(1, 1), :], nsem).wait()

    @pl.when(i + 1 < nsteps)
    def _prefetch():
        base = (i + 1) * rows
        for r in range(rows):
            pltpu.make_async_copy(
                x_any.at[pl.ds(pos_ref[base + r], 1), :],
                gbuf.at[nxt, pl.ds(r, 1), :], gsem).start()

    pltpu.make_async_copy(x_any.at[pl.ds(0, rows), :],
                          gbuf.at[slot], gsem).wait()

    a = x_blk[...]
    g = gbuf[slot]
    tgt = tgt_ref[...]
    is_diff = tgt != meta_ref[1]
    neg = jnp.where(is_diff, negrows[0:1, :], negrows[1:2, :])
    s = a + g + neg
    m = jnp.max(s, axis=1, keepdims=True)
    z = jnp.sum(jnp.exp(s - m), axis=1, keepdims=True)
    logz = m + jnp.log(z)
    lane = jax.lax.broadcasted_iota(jnp.int32, (rows, cols), 1)
    picked = jnp.sum(jnp.where(lane == tgt, s, 0.0), axis=1, keepdims=True)
    v = valid_ref[...]
    acc[0] += jnp.sum(v * (logz - picked))
    acc[1] += jnp.sum(v)

    @pl.when(i + 1 == nsteps)
    def _fin():
        out_ref[0, 0] = acc[0]
        out_ref[0, 1] = acc[1]


def _loss3d_body(meta_ref, a_any, a_blk, y_blk, tgt_ref, valid_ref, out_ref,
                 negrows, acc, nsem, *, rows):
    i = pl.program_id(0)
    nsteps = pl.num_programs(0)

    @pl.when(i == 0)
    def _init():
        acc[0] = 0.0
        acc[1] = 0.0
        pltpu.make_async_copy(a_any.at[pl.ds(0, 1)],
                              negrows.at[pl.ds(0, 1)], nsem).start()
        pltpu.make_async_copy(a_any.at[pl.ds(meta_ref[0], 1)],
                              negrows.at[pl.ds(1, 1)], nsem).start()
        pltpu.make_async_copy(a_any.at[pl.ds(0, 1)],
                              negrows.at[pl.ds(0, 1)], nsem).wait()
        pltpu.make_async_copy(a_any.at[pl.ds(0, 1)],
                              negrows.at[pl.ds(1, 1)], nsem).wait()

    a = a_blk[...]
    g = y_blk[...]
    tgt = tgt_ref[...]
    is_diff = tgt != meta_ref[1]
    neg = jnp.where(is_diff, negrows[0:1], negrows[1:2])
    s = a + g + neg
    m = jnp.max(s, axis=(1, 2), keepdims=True)
    z = jnp.sum(jnp.exp(s - m), axis=(1, 2), keepdims=True)
    logz = (m + jnp.log(z)).reshape(rows, 1)
    sub = jax.lax.broadcasted_iota(jnp.int32, (rows, 8, 128), 1)
    lane = jax.lax.broadcasted_iota(jnp.int32, (rows, 8, 128), 2)
    col = sub * 128 + lane
    picked = jnp.sum(jnp.where(col == tgt, s, 0.0), axis=(1, 2),
                     keepdims=True).reshape(rows, 1)
    v = valid_ref[...].reshape(rows, 1)
    acc[0] += jnp.sum(v * (logz - picked))
    acc[1] += jnp.sum(v)

    @pl.when(i + 1 == nsteps)
    def _fin():
        out_ref[0, 0] = acc[0]
        out_ref[0, 1] = acc[1]


@jax.jit
def kernel(x, target):
    B, C = x.shape
    rows = 256
    half = B // 2
    n_workers = 32
    bands = B // 8
    bands_per_w = bands // n_workers
    n_full = C // 128

    positive, inv, valid, j1, l0 = _triplet_indices(target)
    meta = jnp.stack([jnp.maximum(j1, 0), l0]).astype(jnp.int32)
    tgt2d = target.reshape(B, 1).astype(jnp.int32)
    valid2d = valid.reshape(B, 1).astype(jnp.float32)
    tgt3d = target.reshape(B, 1, 1).astype(jnp.int32)
    valid3d = valid.reshape(B, 1, 1).astype(jnp.float32)

    xs_p = jnp.pad(x[:, n_full * 128:], ((0, 0), (0, 128 * (n_full + 1) - C)),
                   constant_values=-jnp.inf)
    e = jnp.arange(64, dtype=jnp.int32)
    t_e, r_e = e // 8, e % 8
    b = jnp.arange(bands, dtype=jnp.int32)
    k_src = b[:, None] * 8 + r_e[None, :]
    sidx_a = (k_src * 8 + t_e[None, :]).reshape(n_workers, bands_per_w, 64)
    k_dst = inv.reshape(bands, 8)[:, r_e]
    sidx_y = (k_dst * 8 + t_e[None, :]).reshape(n_workers, bands_per_w, 64)
    sc_scatter = _make_sc_scatter(B, C, n_workers, bands_per_w)
    a_tab, y_tab = sc_scatter(x, xs_p, sidx_a, sidx_y)
    a3 = a_tab.reshape(B, 8, 128)
    y3 = y_tab.reshape(B, 8, 128)

    grid1 = pltpu.PrefetchScalarGridSpec(
        num_scalar_prefetch=2,
        grid=(half // rows,),
        in_specs=[
            pl.BlockSpec(memory_space=pltpu.MemorySpace.HBM),
            pl.BlockSpec((rows, C), lambda i, m, p: (i, 0)),
            pl.BlockSpec((rows, 1), lambda i, m, p: (i, 0)),
            pl.BlockSpec((rows, 1), lambda i, m, p: (i, 0)),
        ],
        out_specs=pl.BlockSpec(memory_space=pltpu.MemorySpace.SMEM),
        scratch_shapes=[
            pltpu.VMEM((2, rows, C), jnp.float32),
            pltpu.VMEM((2, C), jnp.float32),
            pltpu.SMEM((2,), jnp.float32),
            pltpu.SemaphoreType.DMA,
            pltpu.SemaphoreType.DMA,
        ],
    )
    out1 = pl.pallas_call(
        functools.partial(_loss2d_body, rows=rows, cols=C),
        grid_spec=grid1,
        out_shape=jax.ShapeDtypeStruct((1, 2), jnp.float32),
    )(meta, positive[:half], x, x, tgt2d[:half], valid2d[:half])

    off = half // rows
    grid2 = pltpu.PrefetchScalarGridSpec(
        num_scalar_prefetch=1,
        grid=(half // rows,),
        in_specs=[
            pl.BlockSpec(memory_space=pltpu.MemorySpace.HBM),
            pl.BlockSpec((rows, 8, 128), lambda i, m: (i + off, 0, 0)),
            pl.BlockSpec((rows, 8, 128), lambda i, m: (i + off, 0, 0)),
            pl.BlockSpec((rows, 1, 1), lambda i, m: (i + off, 0, 0)),
            pl.BlockSpec((rows, 1, 1), lambda i, m: (i + off, 0, 0)),
        ],
        out_specs=pl.BlockSpec(memory_space=pltpu.MemorySpace.SMEM),
        scratch_shapes=[
            pltpu.VMEM((2, 8, 128), jnp.float32),
            pltpu.SMEM((2,), jnp.float32),
            pltpu.SemaphoreType.DMA,
        ],
    )
    out2 = pl.pallas_call(
        functools.partial(_loss3d_body, rows=rows),
        grid_spec=grid2,
        out_shape=jax.ShapeDtypeStruct((1, 2), jnp.float32),
    )(meta, a3, a3, y3, tgt3d, valid3d)

    s = out1[0, 0] + out2[0, 0]
    c = out1[0, 1] + out2[0, 1]
    return s / c


# speedup vs baseline: 1.4135x; 1.4135x over previous
import functools

import jax
import jax.numpy as jnp
from jax.experimental import pallas as pl
from jax.experimental.pallas import tpu as pltpu


def _triplet_indices(target):
    B = target.shape[0]
    idx = jnp.arange(B, dtype=jnp.int32)
    t = target.astype(jnp.int32)
    skey = jnp.sort(t * B + idx)
    order = skey % B
    sorted_lbl = skey // B
    new_group = jnp.concatenate(
        [jnp.array([True]), sorted_lbl[1:] != sorted_lbl[:-1]])
    starts_per_pos = jax.lax.cummax(jnp.where(new_group, idx, 0))
    flagged = jnp.where(new_group, idx, B)
    rev_min = jax.lax.cummin(flagged, reverse=True)
    next_start = jnp.concatenate([rev_min[1:], jnp.array([B], rev_min.dtype)])
    counts = next_start - starts_per_pos
    pos_within = idx - starts_per_pos
    partner_sorted = starts_per_pos + (pos_within + 1) % counts
    positive = jnp.zeros(B, jnp.int32).at[order].set(order[partner_sorted])

    l0 = t[0]
    diff = t != l0
    j1 = jnp.where(jnp.any(diff), jnp.argmax(diff).astype(jnp.int32),
                   jnp.int32(-1))
    valid = (positive != idx) & (diff | (j1 >= 0))
    return positive, valid, j1, l0


def _loss_body(meta_ref, pos_ref, x_any, x_blk, tgt_ref, valid_ref, out_ref,
               gbuf, negrows, acc, gsem, nsem, *, rows, cols):
    i = pl.program_id(0)
    nsteps = pl.num_programs(0)
    slot = jax.lax.rem(i, 2)
    nxt = 1 - slot

    @pl.when(i == 0)
    def _init():
        acc[0] = 0.0
        acc[1] = 0.0
        pltpu.make_async_copy(x_any.at[pl.ds(0, 1), :],
                              negrows.at[pl.ds(0, 1), :], nsem).start()
        pltpu.make_async_copy(x_any.at[pl.ds(meta_ref[0], 1), :],
                              negrows.at[pl.ds(1, 1), :], nsem).start()
        for r in range(rows):
            pltpu.make_async_copy(
                x_any.at[pl.ds(pos_ref[r], 1), :],
                gbuf.at[slot, pl.ds(r, 1), :], gsem).start()
        pltpu.make_async_copy(x_any.at[pl.ds(0, 1), :],
                              negrows.at[pl.ds(0, 1), :], nsem).wait()
        pltpu.make_async_copy(x_any.at[pl.ds(0, 1), :],
                              negrows.at[pl.ds(1, 1), :], nsem).wait()

    @pl.when(i + 1 < nsteps)
    def _prefetch():
        base = (i + 1) * rows
        for r in range(rows):
            pltpu.make_async_copy(
                x_any.at[pl.ds(pos_ref[base + r], 1), :],
                gbuf.at[nxt, pl.ds(r, 1), :], gsem).start()

    pltpu.make_async_copy(x_any.at[pl.ds(0, rows), :],
                          gbuf.at[slot], gsem).wait()

    a = x_blk[...]
    g = gbuf[slot]
    tgt = tgt_ref[...]
    is_diff = tgt != meta_ref[1]
    neg = jnp.where(is_diff, negrows[0:1, :], negrows[1:2, :])
    s = a + g + neg
    m = jnp.max(s, axis=1, keepdims=True)
    z = jnp.sum(jnp.exp(s - m), axis=1, keepdims=True)
    logz = m + jnp.log(z)
    lane = jax.lax.broadcasted_iota(jnp.int32, (rows, cols), 1)
    picked = jnp.sum(jnp.where(lane == tgt, s, 0.0), axis=1, keepdims=True)
    v = valid_ref[...]
    acc[0] += jnp.sum(v * (logz - picked))
    acc[1] += jnp.sum(v)

    @pl.when(i + 1 == nsteps)
    def _fin():
        out_ref[0, 0] = acc[0] / acc[1]


@jax.jit
def kernel(x, target):
    B, C = x.shape
    rows = 256
    nsteps = B // rows

    positive, valid, j1, l0 = _triplet_indices(target)
    meta = jnp.stack([jnp.maximum(j1, 0), l0]).astype(jnp.int32)
    tgt2d = target.reshape(B, 1).astype(jnp.int32)
    valid2d = valid.reshape(B, 1).astype(jnp.float32)

    grid_spec = pltpu.PrefetchScalarGridSpec(
        num_scalar_prefetch=2,
        grid=(nsteps,),
        in_specs=[
            pl.BlockSpec(memory_space=pltpu.MemorySpace.HBM),
            pl.BlockSpec((rows, C), lambda i, m, p: (i, 0)),
            pl.BlockSpec((rows, 1), lambda i, m, p: (i, 0)),
            pl.BlockSpec((rows, 1), lambda i, m, p: (i, 0)),
        ],
        out_specs=pl.BlockSpec(memory_space=pltpu.MemorySpace.SMEM),
        scratch_shapes=[
            pltpu.VMEM((2, rows, C), jnp.float32),
            pltpu.VMEM((2, C), jnp.float32),
            pltpu.SMEM((2,), jnp.float32),
            pltpu.SemaphoreType.DMA,
            pltpu.SemaphoreType.DMA,
        ],
    )
    out = pl.pallas_call(
        functools.partial(_loss_body, rows=rows, cols=C),
        grid_spec=grid_spec,
        out_shape=jax.ShapeDtypeStruct((1, 1), jnp.float32),
    )(meta, positive, x, x, tgt2d, valid2d)
    return out.reshape(())


# speedup vs baseline: 1.4352x; 1.0154x over previous
import functools

import jax
import jax.numpy as jnp
from jax.experimental import pallas as pl
from jax.experimental.pallas import tpu as pltpu


def _triplet_indices(target):
    B = target.shape[0]
    idx = jnp.arange(B, dtype=jnp.int32)
    t = target.astype(jnp.int32)
    skey = jnp.sort(t * B + idx)
    order = skey % B
    sorted_lbl = skey // B
    new_group = jnp.concatenate(
        [jnp.array([True]), sorted_lbl[1:] != sorted_lbl[:-1]])
    starts_per_pos = jax.lax.cummax(jnp.where(new_group, idx, 0))
    flagged = jnp.where(new_group, idx, B)
    rev_min = jax.lax.cummin(flagged, reverse=True)
    next_start = jnp.concatenate([rev_min[1:], jnp.array([B], rev_min.dtype)])
    counts = next_start - starts_per_pos
    pos_within = idx - starts_per_pos
    partner_sorted = starts_per_pos + (pos_within + 1) % counts
    positive = jnp.zeros(B, jnp.int32).at[order].set(order[partner_sorted])

    l0 = t[0]
    diff = t != l0
    j1 = jnp.where(jnp.any(diff), jnp.argmax(diff).astype(jnp.int32),
                   jnp.int32(-1))
    valid = (positive != idx) & (diff | (j1 >= 0))
    return positive, valid, j1, l0


def _loss_body(meta_ref, pos_ref, x_any, x_blk, tgt_ref, valid_ref, out_ref,
               gbuf, negrows, acc, gsem, nsem, *, rows, cols):
    i = pl.program_id(0)
    nsteps = pl.num_programs(0)
    slot = jax.lax.rem(i, 2)
    nxt = 1 - slot

    @pl.when(i == 0)
    def _init():
        acc[0] = 0.0
        acc[1] = 0.0
        pltpu.make_async_copy(x_any.at[pl.ds(0, 1), :],
                              negrows.at[pl.ds(0, 1), :], nsem).start()
        pltpu.make_async_copy(x_any.at[pl.ds(meta_ref[0], 1), :],
                              negrows.at[pl.ds(1, 1), :], nsem).start()
        for r in range(rows):
            pltpu.make_async_copy(
                x_any.at[pl.ds(pos_ref[r], 1), :],
                gbuf.at[slot, pl.ds(r, 1), :], gsem).start()
        pltpu.make_async_copy(x_any.at[pl.ds(0, 1), :],
                              negrows.at[pl.ds(0, 1), :], nsem).wait()
        pltpu.make_async_copy(x_any.at[pl.ds(0, 1), :],
                              negrows.at[pl.ds(1, 1), :], nsem).wait()

    @pl.when(i + 1 < nsteps)
    def _prefetch():
        base = (i + 1) * rows
        for r in range(rows):
            pltpu.make_async_copy(
                x_any.at[pl.ds(pos_ref[base + r], 1), :],
                gbuf.at[nxt, pl.ds(r, 1), :], gsem).start()

    pltpu.make_async_copy(x_any.at[pl.ds(0, rows), :],
                          gbuf.at[slot], gsem).wait()

    a = x_blk[...]
    g = gbuf[slot]
    tgt = tgt_ref[...]
    is_diff = tgt != meta_ref[1]
    neg = jnp.where(is_diff, negrows[0:1, :], negrows[1:2, :])
    s = a + g + neg
    m = jnp.max(s, axis=1, keepdims=True)
    z = jnp.sum(jnp.exp(s - m), axis=1, keepdims=True)
    logz = m + jnp.log(z)
    lane = jax.lax.broadcasted_iota(jnp.int32, (rows, cols), 1)
    picked = jnp.sum(jnp.where(lane == tgt, s, 0.0), axis=1, keepdims=True)
    v = valid_ref[...]
    acc[0] += jnp.sum(v * (logz - picked))
    acc[1] += jnp.sum(v)

    @pl.when(i + 1 == nsteps)
    def _fin():
        out_ref[0, 0] = acc[0] / acc[1]


@jax.jit
def kernel(x, target):
    B, C = x.shape
    rows = 512
    nsteps = B // rows

    positive, valid, j1, l0 = _triplet_indices(target)
    meta = jnp.stack([jnp.maximum(j1, 0), l0]).astype(jnp.int32)
    tgt2d = target.reshape(B, 1).astype(jnp.int32)
    valid2d = valid.reshape(B, 1).astype(jnp.float32)

    grid_spec = pltpu.PrefetchScalarGridSpec(
        num_scalar_prefetch=2,
        grid=(nsteps,),
        in_specs=[
            pl.BlockSpec(memory_space=pltpu.MemorySpace.HBM),
            pl.BlockSpec((rows, C), lambda i, m, p: (i, 0)),
            pl.BlockSpec((rows, 1), lambda i, m, p: (i, 0)),
            pl.BlockSpec((rows, 1), lambda i, m, p: (i, 0)),
        ],
        out_specs=pl.BlockSpec(memory_space=pltpu.MemorySpace.SMEM),
        scratch_shapes=[
            pltpu.VMEM((2, rows, C), jnp.float32),
            pltpu.VMEM((2, C), jnp.float32),
            pltpu.SMEM((2,), jnp.float32),
            pltpu.SemaphoreType.DMA,
            pltpu.SemaphoreType.DMA,
        ],
    )
    out = pl.pallas_call(
        functools.partial(_loss_body, rows=rows, cols=C),
        grid_spec=grid_spec,
        out_shape=jax.ShapeDtypeStruct((1, 1), jnp.float32),
    )(meta, positive, x, x, tgt2d, valid2d)
    return out.reshape(())
